# Initial kernel scaffold; baseline (speedup 1.0000x reference)
import functools

import jax
import jax.numpy as jnp
from jax import lax
from jax.experimental import pallas as pl
from jax.experimental.pallas import tpu as pltpu
from jax.experimental.pallas import tpu_sc as plsc

N = 10000
E = 320000
D_IN = 128
D_OUT = 64

NC = 2
NS = 16
NW = NC * NS
L = 16

DH = 80
N_PAD = 10240
ROWS_PER_SUB = N_PAD // NS

C = 128
E_TOT = E + N
CHUNKS = -(-E_TOT // (NW * C))
E_PAD = NW * C * CHUNKS
EDGES_PER_TILE = C * CHUNKS


def _dense_body(x_ref, w_ref, asrc_ref, adst_ref, hext_ref, sa_ref, sd_ref):
    i = pl.program_id(0)
    h = jnp.dot(x_ref[...], w_ref[...], preferred_element_type=jnp.float32)
    blk = x_ref.shape[0]
    row = i * blk + lax.broadcasted_iota(jnp.int32, (blk, 1), 0)
    flag = (row < N).astype(jnp.float32)
    hext_ref[:, :D_OUT] = h
    hext_ref[:, D_OUT:D_OUT + 1] = flag
    hext_ref[:, D_OUT + 1:] = jnp.zeros((blk, DH - D_OUT - 1), jnp.float32)
    sa_ref[...] = jnp.sum(h * asrc_ref[...], axis=1, keepdims=True)
    sd_ref[...] = jnp.sum(h * adst_ref[...], axis=1, keepdims=True)


def _dense_prologue(x_pad, W, att_src, att_dst):
    blk = 1024
    grid = N_PAD // blk
    return pl.pallas_call(
        _dense_body,
        grid=(grid,),
        in_specs=[
            pl.BlockSpec((blk, D_IN), lambda i: (i, 0)),
            pl.BlockSpec((D_IN, D_OUT), lambda i: (0, 0)),
            pl.BlockSpec((1, D_OUT), lambda i: (0, 0)),
            pl.BlockSpec((1, D_OUT), lambda i: (0, 0)),
        ],
        out_specs=[
            pl.BlockSpec((blk, DH), lambda i: (i, 0)),
            pl.BlockSpec((blk, 1), lambda i: (i, 0)),
            pl.BlockSpec((blk, 1), lambda i: (i, 0)),
        ],
        out_shape=[
            jax.ShapeDtypeStruct((N_PAD, DH), jnp.float32),
            jax.ShapeDtypeStruct((N_PAD, 1), jnp.float32),
            jax.ShapeDtypeStruct((N_PAD, 1), jnp.float32),
        ],
    )(x_pad, W, att_src, att_dst)


def _sc_body(hext_hbm, asrc_hbm, adst_hbm, src_hbm, dst_hbm, out_hbm,
             asrc_l, adst_l, src_v, dst_v, ex_v, rows_v, acc_sh, sem):
    c = lax.axis_index("c")
    s = lax.axis_index("s")
    wid = s * NC + c

    pltpu.sync_copy(asrc_hbm, asrc_l)
    pltpu.sync_copy(adst_hbm, adst_l)

    @pl.loop(0, C)
    def _(e):
        for j in range(DH // L):
            rows_v[e, pl.ds(j * L, L)] = jnp.zeros((L,), jnp.float32)

    for k in range(ROWS_PER_SUB // C):
        pltpu.sync_copy(rows_v, acc_sh.at[pl.ds(s * ROWS_PER_SUB + k * C, C)])
    plsc.subcore_barrier()

    base = wid * EDGES_PER_TILE

    @pl.loop(0, CHUNKS)
    def _(k):
        off = base + k * C
        pltpu.sync_copy(src_hbm.at[pl.ds(off, C)], src_v)
        pltpu.sync_copy(dst_hbm.at[pl.ds(off, C)], dst_v)
        pltpu.async_copy(hext_hbm.at[src_v], rows_v, sem).wait()

        for g in range(C // L):
            sl = pl.ds(g * L, L)
            si = src_v[sl]
            di = dst_v[sl]
            av = plsc.load_gather(asrc_l, [si]) + plsc.load_gather(adst_l, [di])
            av = jnp.maximum(av, av * 0.2)
            ex_v[sl] = jnp.exp(av)

        @pl.loop(0, C)
        def _(e):
            cvec = jnp.full((L,), ex_v[e], jnp.float32)
            for j in range(DH // L):
                sl = pl.ds(j * L, L)
                rows_v[e, sl] = rows_v[e, sl] * cvec

        pltpu.sync_copy(rows_v, acc_sh.at[dst_v], add=True)

    plsc.subcore_barrier()
    pltpu.sync_copy(acc_sh.at[pl.ds(s * ROWS_PER_SUB, ROWS_PER_SUB)],
                    out_hbm.at[c, pl.ds(s * ROWS_PER_SUB, ROWS_PER_SUB)])


def _sc_scatter(h_ext, asrc, adst, src_all, dst_all):
    mesh = plsc.VectorSubcoreMesh(core_axis_name="c", subcore_axis_name="s")
    f = pl.kernel(
        _sc_body,
        out_type=jax.ShapeDtypeStruct((NC, N_PAD, DH), jnp.float32),
        mesh=mesh,
        scratch_types=[
            pltpu.VMEM((N_PAD,), jnp.float32),
            pltpu.VMEM((N_PAD,), jnp.float32),
            pltpu.VMEM((C,), jnp.int32),
            pltpu.VMEM((C,), jnp.int32),
            pltpu.VMEM((C,), jnp.float32),
            pltpu.VMEM((C, DH), jnp.float32),
            pltpu.VMEM_SHARED((N_PAD, DH), jnp.float32),
            pltpu.SemaphoreType.DMA,
        ],
    )
    return f(h_ext, asrc, adst, src_all, dst_all)


def _final_body(p0_ref, p1_ref, b_ref, o_ref):
    ssum = p0_ref[...] + p1_ref[...]
    num = ssum[:, :D_OUT]
    den = ssum[:, D_OUT:D_OUT + 1]
    o = num / (den + 1e-16) + b_ref[...]
    o_ref[...] = jnp.where(o > 0, o, jnp.expm1(o))


def _finalize(p0, p1, bias):
    blk = 1000
    grid = N // blk
    return pl.pallas_call(
        _final_body,
        grid=(grid,),
        in_specs=[
            pl.BlockSpec((blk, DH), lambda i: (i, 0)),
            pl.BlockSpec((blk, DH), lambda i: (i, 0)),
            pl.BlockSpec((1, D_OUT), lambda i: (0, 0)),
        ],
        out_specs=pl.BlockSpec((blk, D_OUT), lambda i: (i, 0)),
        out_shape=jax.ShapeDtypeStruct((N, D_OUT), jnp.float32),
    )(p0, p1, bias)


def kernel(x, edge_index, W, att_src, att_dst, bias):
    loop = jnp.arange(N, dtype=jnp.int32)
    pad = jnp.full((E_PAD - E_TOT,), N, dtype=jnp.int32)
    src_all = jnp.concatenate([edge_index[0], loop, pad])
    dst_all = jnp.concatenate([edge_index[1], loop, pad])

    x_pad = jnp.concatenate(
        [x, jnp.zeros((N_PAD - N, D_IN), jnp.float32)], axis=0)
    h_ext, sa, sd = _dense_prologue(
        x_pad, W, att_src.reshape(1, D_OUT), att_dst.reshape(1, D_OUT))

    partials = _sc_scatter(h_ext, sa.reshape(N_PAD), sd.reshape(N_PAD),
                           src_all, dst_all)

    return _finalize(partials[0, :N, :], partials[1, :N, :],
                     bias.reshape(1, D_OUT))


# speedup vs baseline: 28.1104x; 28.1104x over previous
import dataclasses
import functools

import jax
import jax.numpy as jnp
from jax import lax
from jax.experimental import pallas as pl
from jax.experimental.pallas import tpu as pltpu
from jax.experimental.pallas import tpu_sc as plsc

N = 10000
E = 320000
D_IN = 128
D_OUT = 64

NC = 2
NS = 16
NW = NC * NS
L = 16

DH = 80
N_PAD = 10240
ROWS_PER_SUB = N_PAD // NS

C = 128
E_TOT = E + N
CHUNKS = -(-E_TOT // (NW * C))
E_PAD = NW * C * CHUNKS
EDGES_PER_TILE = C * CHUNKS


def _dense_body(x_ref, w_ref, asrc_ref, adst_ref, hext_ref, sa_ref, sd_ref):
    i = pl.program_id(0)
    h = jnp.dot(x_ref[...], w_ref[...], preferred_element_type=jnp.float32)
    blk = x_ref.shape[0]
    row = i * blk + lax.broadcasted_iota(jnp.int32, (blk, 1), 0)
    flag = (row < N).astype(jnp.float32)
    hext_ref[:, :D_OUT] = h
    hext_ref[:, D_OUT:D_OUT + 1] = flag
    hext_ref[:, D_OUT + 1:] = jnp.zeros((blk, DH - D_OUT - 1), jnp.float32)
    sa_ref[...] = jnp.sum(h * asrc_ref[...], axis=1, keepdims=True)
    sd_ref[...] = jnp.sum(h * adst_ref[...], axis=1, keepdims=True)


def _dense_prologue(x_pad, W, att_src, att_dst):
    blk = 1024
    grid = N_PAD // blk
    return pl.pallas_call(
        _dense_body,
        grid=(grid,),
        in_specs=[
            pl.BlockSpec((blk, D_IN), lambda i: (i, 0)),
            pl.BlockSpec((D_IN, D_OUT), lambda i: (0, 0)),
            pl.BlockSpec((1, D_OUT), lambda i: (0, 0)),
            pl.BlockSpec((1, D_OUT), lambda i: (0, 0)),
        ],
        out_specs=[
            pl.BlockSpec((blk, DH), lambda i: (i, 0)),
            pl.BlockSpec((blk, 1), lambda i: (i, 0)),
            pl.BlockSpec((blk, 1), lambda i: (i, 0)),
        ],
        out_shape=[
            jax.ShapeDtypeStruct((N_PAD, DH), jnp.float32),
            jax.ShapeDtypeStruct((N_PAD, 1), jnp.float32),
            jax.ShapeDtypeStruct((N_PAD, 1), jnp.float32),
        ],
    )(x_pad, W, att_src, att_dst)


def _sc_body(hext_hbm, asrc_hbm, adst_hbm, src_hbm, dst_hbm, out_hbm,
             asrc_l, adst_l, src_v, dst_v, ex_v, rows_v, acc_sh, sem):
    c = lax.axis_index("c")
    s = lax.axis_index("s")
    wid = s * NC + c

    pltpu.sync_copy(asrc_hbm, asrc_l)
    pltpu.sync_copy(adst_hbm, adst_l)

    @pl.loop(0, C)
    def _(e):
        for j in range(DH // L):
            rows_v[e, pl.ds(j * L, L)] = jnp.zeros((L,), jnp.float32)

    for k in range(ROWS_PER_SUB // C):
        pltpu.sync_copy(rows_v, acc_sh.at[pl.ds(s * ROWS_PER_SUB + k * C, C)])
    plsc.subcore_barrier()

    base = wid * EDGES_PER_TILE

    @pl.loop(0, CHUNKS)
    def _(k):
        off = base + k * C
        pltpu.sync_copy(src_hbm.at[pl.ds(off, C)], src_v)
        pltpu.sync_copy(dst_hbm.at[pl.ds(off, C)], dst_v)
        pltpu.async_copy(hext_hbm.at[src_v], rows_v, sem).wait()

        for g in range(C // L):
            sl = pl.ds(g * L, L)
            si = src_v[sl]
            di = dst_v[sl]
            av = plsc.load_gather(asrc_l, [si]) + plsc.load_gather(adst_l, [di])
            av = jnp.maximum(av, av * 0.2)
            ex_v[sl] = jnp.exp(av)

        @pl.loop(0, C)
        def _(e):
            ev = ex_v[pl.ds(e, L)]
            cvec = jnp.full((L,), ev[0], jnp.float32)
            for j in range(DH // L):
                sl = pl.ds(j * L, L)
                rows_v[e, sl] = rows_v[e, sl] * cvec

        pltpu.sync_copy(rows_v, acc_sh.at[dst_v], add=True)

    plsc.subcore_barrier()
    pltpu.sync_copy(acc_sh.at[pl.ds(s * ROWS_PER_SUB, ROWS_PER_SUB)],
                    out_hbm.at[c, pl.ds(s * ROWS_PER_SUB, ROWS_PER_SUB)])


def _sc_scatter(h_ext, asrc, adst, src_all, dst_all):
    mesh = plsc.VectorSubcoreMesh(core_axis_name="c", subcore_axis_name="s")
    cp = pltpu.CompilerParams(needs_layout_passes=False,
                              use_tc_tiling_on_sc=False)
    f = pl.kernel(
        _sc_body,
        compiler_params=cp,
        out_type=jax.ShapeDtypeStruct((NC, N_PAD, DH), jnp.float32),
        mesh=mesh,
        scratch_types=[
            pltpu.VMEM((N_PAD,), jnp.float32),
            pltpu.VMEM((N_PAD,), jnp.float32),
            pltpu.VMEM((C,), jnp.int32),
            pltpu.VMEM((C,), jnp.int32),
            pltpu.VMEM((C + L,), jnp.float32),
            pltpu.VMEM((C, DH), jnp.float32),
            pltpu.VMEM_SHARED((N_PAD, DH), jnp.float32),
            pltpu.SemaphoreType.DMA,
        ],
    )
    return f(h_ext, asrc, adst, src_all, dst_all)


def _final_body(p0_ref, p1_ref, b_ref, o_ref):
    ssum = p0_ref[...] + p1_ref[...]
    num = ssum[:, :D_OUT]
    den = ssum[:, D_OUT:D_OUT + 1]
    o = num / (den + 1e-16) + b_ref[...]
    o_ref[...] = jnp.where(o > 0, o, jnp.exp(o) - 1.0)


def _finalize(p0, p1, bias):
    blk = 1000
    grid = N // blk
    return pl.pallas_call(
        _final_body,
        grid=(grid,),
        in_specs=[
            pl.BlockSpec((blk, DH), lambda i: (i, 0)),
            pl.BlockSpec((blk, DH), lambda i: (i, 0)),
            pl.BlockSpec((1, D_OUT), lambda i: (0, 0)),
        ],
        out_specs=pl.BlockSpec((blk, D_OUT), lambda i: (i, 0)),
        out_shape=jax.ShapeDtypeStruct((N, D_OUT), jnp.float32),
    )(p0, p1, bias)


def kernel(x, edge_index, W, att_src, att_dst, bias):
    loop = jnp.arange(N, dtype=jnp.int32)
    pad = jnp.full((E_PAD - E_TOT,), N, dtype=jnp.int32)
    src_all = jnp.concatenate([edge_index[0], loop, pad])
    dst_all = jnp.concatenate([edge_index[1], loop, pad])

    x_pad = jnp.concatenate(
        [x, jnp.zeros((N_PAD - N, D_IN), jnp.float32)], axis=0)
    h_ext, sa, sd = _dense_prologue(
        x_pad, W, att_src.reshape(1, D_OUT), att_dst.reshape(1, D_OUT))

    partials = _sc_scatter(h_ext, sa.reshape(N_PAD), sd.reshape(N_PAD),
                           src_all, dst_all)

    return _finalize(partials[0, :N, :], partials[1, :N, :],
                     bias.reshape(1, D_OUT))


# speedup vs baseline: 33.0344x; 1.1752x over previous
import jax
import jax.numpy as jnp
from jax import lax
from jax.experimental import pallas as pl
from jax.experimental.pallas import tpu as pltpu
from jax.experimental.pallas import tpu_sc as plsc

N = 10000
E = 320000
D_IN = 128
D_OUT = 64

NC = 2
NS = 16
NW = NC * NS
L = 16

DH = 80
N_PAD = 10240
ROWS_PER_SUB = N_PAD // NS

C = 128
E_TOT = E + N
CHUNKS = 82
E_PAD = NW * C * CHUNKS
EDGES_PER_TILE = C * CHUNKS


def _dense_body(x_ref, w_ref, asrc_ref, adst_ref, hext_ref, sa_ref, sd_ref):
    i = pl.program_id(0)
    h = jnp.dot(x_ref[...], w_ref[...], preferred_element_type=jnp.float32)
    blk = x_ref.shape[0]
    row = i * blk + lax.broadcasted_iota(jnp.int32, (blk, 1), 0)
    flag = (row < N).astype(jnp.float32)
    hext_ref[:, :D_OUT] = h
    hext_ref[:, D_OUT:D_OUT + 1] = flag
    hext_ref[:, D_OUT + 1:] = jnp.zeros((blk, DH - D_OUT - 1), jnp.float32)
    sa_ref[...] = jnp.sum(h * asrc_ref[...], axis=1, keepdims=True)
    sd_ref[...] = jnp.sum(h * adst_ref[...], axis=1, keepdims=True)


def _dense_prologue(x_pad, W, att_src, att_dst):
    blk = 1024
    grid = N_PAD // blk
    return pl.pallas_call(
        _dense_body,
        grid=(grid,),
        in_specs=[
            pl.BlockSpec((blk, D_IN), lambda i: (i, 0)),
            pl.BlockSpec((D_IN, D_OUT), lambda i: (0, 0)),
            pl.BlockSpec((1, D_OUT), lambda i: (0, 0)),
            pl.BlockSpec((1, D_OUT), lambda i: (0, 0)),
        ],
        out_specs=[
            pl.BlockSpec((blk, DH), lambda i: (i, 0)),
            pl.BlockSpec((blk, 1), lambda i: (i, 0)),
            pl.BlockSpec((blk, 1), lambda i: (i, 0)),
        ],
        out_shape=[
            jax.ShapeDtypeStruct((N_PAD, DH), jnp.float32),
            jax.ShapeDtypeStruct((N_PAD, 1), jnp.float32),
            jax.ShapeDtypeStruct((N_PAD, 1), jnp.float32),
        ],
    )(x_pad, W, att_src, att_dst)


def _sc_body(hext_hbm, asrc_hbm, adst_hbm, src_hbm, dst_hbm, out_hbm,
             asrc_l, adst_l, src_a, dst_a, src_b, dst_b,
             rows_a, rows_b, acc_sh, sem_a, sem_b):
    c = lax.axis_index("c")
    s = lax.axis_index("s")
    wid = s * NC + c

    pltpu.sync_copy(asrc_hbm, asrc_l)
    pltpu.sync_copy(adst_hbm, adst_l)

    @pl.loop(0, C)
    def _(e):
        for j in range(DH // L):
            rows_a[e, pl.ds(j * L, L)] = jnp.zeros((L,), jnp.float32)

    for k in range(ROWS_PER_SUB // C):
        pltpu.sync_copy(rows_a, acc_sh.at[pl.ds(s * ROWS_PER_SUB + k * C, C)])
    plsc.subcore_barrier()

    base = wid * EDGES_PER_TILE

    def load_idx(k, sv, dv):
        off = base + k * C
        pltpu.sync_copy(src_hbm.at[pl.ds(off, C)], sv)
        pltpu.sync_copy(dst_hbm.at[pl.ds(off, C)], dv)

    dnums = lax.GatherDimensionNumbers(
        offset_dims=(), collapsed_slice_dims=(0,), start_index_map=(0,))

    def process(sv, dv, rows):
        for g in range(C // L):
            sl = pl.ds(g * L, L)
            si = sv[sl]
            di = dv[sl]
            av = plsc.load_gather(asrc_l, [si]) + plsc.load_gather(adst_l, [di])
            av = jnp.maximum(av, av * 0.2)
            exg = jnp.exp(av)
            for t in range(L):
                bc = lax.gather(
                    exg, jnp.full((L, 1), t, jnp.int32), dnums, (1,),
                    mode=lax.GatherScatterMode.PROMISE_IN_BOUNDS)
                row = g * L + t
                for j in range(DH // L):
                    sl2 = pl.ds(j * L, L)
                    rows[row, sl2] = rows[row, sl2] * bc
        pltpu.sync_copy(rows, acc_sh.at[dv], add=True)

    load_idx(0, src_a, dst_a)
    pltpu.async_copy(hext_hbm.at[src_a], rows_a, sem_a)

    @pl.loop(0, CHUNKS // 2)
    def _(k2):
        k = 2 * k2
        load_idx(k + 1, src_b, dst_b)
        pltpu.async_copy(hext_hbm.at[src_b], rows_b, sem_b)

        pltpu.make_async_copy(hext_hbm.at[src_a], rows_a, sem_a).wait()
        process(src_a, dst_a, rows_a)

        @pl.when(k2 + 1 < CHUNKS // 2)
        def _():
            load_idx(k + 2, src_a, dst_a)
            pltpu.async_copy(hext_hbm.at[src_a], rows_a, sem_a)

        pltpu.make_async_copy(hext_hbm.at[src_b], rows_b, sem_b).wait()
        process(src_b, dst_b, rows_b)

    plsc.subcore_barrier()
    pltpu.sync_copy(acc_sh.at[pl.ds(s * ROWS_PER_SUB, ROWS_PER_SUB)],
                    out_hbm.at[c, pl.ds(s * ROWS_PER_SUB, ROWS_PER_SUB)])


def _sc_scatter(h_ext, asrc, adst, src_all, dst_all):
    mesh = plsc.VectorSubcoreMesh(core_axis_name="c", subcore_axis_name="s")
    cp = pltpu.CompilerParams(needs_layout_passes=False,
                              use_tc_tiling_on_sc=False)
    f = pl.kernel(
        _sc_body,
        compiler_params=cp,
        out_type=jax.ShapeDtypeStruct((NC, N_PAD, DH), jnp.float32),
        mesh=mesh,
        scratch_types=[
            pltpu.VMEM((N_PAD,), jnp.float32),
            pltpu.VMEM((N_PAD,), jnp.float32),
            pltpu.VMEM((C,), jnp.int32),
            pltpu.VMEM((C,), jnp.int32),
            pltpu.VMEM((C,), jnp.int32),
            pltpu.VMEM((C,), jnp.int32),
            pltpu.VMEM((C, DH), jnp.float32),
            pltpu.VMEM((C, DH), jnp.float32),
            pltpu.VMEM_SHARED((N_PAD, DH), jnp.float32),
            pltpu.SemaphoreType.DMA,
            pltpu.SemaphoreType.DMA,
        ],
    )
    return f(h_ext, asrc, adst, src_all, dst_all)


def _final_body(p0_ref, p1_ref, b_ref, o_ref):
    ssum = p0_ref[...] + p1_ref[...]
    num = ssum[:, :D_OUT]
    den = ssum[:, D_OUT:D_OUT + 1]
    o = num / (den + 1e-16) + b_ref[...]
    o_ref[...] = jnp.where(o > 0, o, jnp.exp(o) - 1.0)


def _finalize(p0, p1, bias):
    blk = 1000
    grid = N // blk
    return pl.pallas_call(
        _final_body,
        grid=(grid,),
        in_specs=[
            pl.BlockSpec((blk, DH), lambda i: (i, 0)),
            pl.BlockSpec((blk, DH), lambda i: (i, 0)),
            pl.BlockSpec((1, D_OUT), lambda i: (0, 0)),
        ],
        out_specs=pl.BlockSpec((blk, D_OUT), lambda i: (i, 0)),
        out_shape=jax.ShapeDtypeStruct((N, D_OUT), jnp.float32),
    )(p0, p1, bias)


def kernel(x, edge_index, W, att_src, att_dst, bias):
    loop = jnp.arange(N, dtype=jnp.int32)
    pad = jnp.full((E_PAD - E_TOT,), N, dtype=jnp.int32)
    src_all = jnp.concatenate([edge_index[0], loop, pad])
    dst_all = jnp.concatenate([edge_index[1], loop, pad])

    x_pad = jnp.concatenate(
        [x, jnp.zeros((N_PAD - N, D_IN), jnp.float32)], axis=0)
    h_ext, sa, sd = _dense_prologue(
        x_pad, W, att_src.reshape(1, D_OUT), att_dst.reshape(1, D_OUT))

    partials = _sc_scatter(h_ext, sa.reshape(N_PAD), sd.reshape(N_PAD),
                           src_all, dst_all)

    return _finalize(partials[0, :N, :], partials[1, :N, :],
                     bias.reshape(1, D_OUT))


# speedup vs baseline: 37.8613x; 1.1461x over previous
import jax
import jax.numpy as jnp
from jax import lax
from jax.experimental import pallas as pl
from jax.experimental.pallas import tpu as pltpu
from jax.experimental.pallas import tpu_sc as plsc

N = 10000
E = 320000
D_IN = 128
D_OUT = 64

NC = 2
NS = 16
NW = NC * NS
L = 16

DH = 64
N_PAD = 10240
ROWS_PER_SUB = N_PAD // NS

C = 128
E_TOT = E + N
CHUNKS = 82
E_PAD = NW * C * CHUNKS
EDGES_PER_TILE = C * CHUNKS


def _dense_body(x_ref, w_ref, asrc_ref, adst_ref, hext_ref, sa_ref, sd_ref):
    h = jnp.dot(x_ref[...], w_ref[...], preferred_element_type=jnp.float32)
    hext_ref[...] = h
    sa_ref[...] = jnp.sum(h * asrc_ref[...], axis=1, keepdims=True)
    sd_ref[...] = jnp.sum(h * adst_ref[...], axis=1, keepdims=True)


def _dense_prologue(x_pad, W, att_src, att_dst):
    blk = 1024
    grid = N_PAD // blk
    return pl.pallas_call(
        _dense_body,
        grid=(grid,),
        in_specs=[
            pl.BlockSpec((blk, D_IN), lambda i: (i, 0)),
            pl.BlockSpec((D_IN, D_OUT), lambda i: (0, 0)),
            pl.BlockSpec((1, D_OUT), lambda i: (0, 0)),
            pl.BlockSpec((1, D_OUT), lambda i: (0, 0)),
        ],
        out_specs=[
            pl.BlockSpec((blk, DH), lambda i: (i, 0)),
            pl.BlockSpec((blk, 1), lambda i: (i, 0)),
            pl.BlockSpec((blk, 1), lambda i: (i, 0)),
        ],
        out_shape=[
            jax.ShapeDtypeStruct((N_PAD, DH), jnp.float32),
            jax.ShapeDtypeStruct((N_PAD, 1), jnp.float32),
            jax.ShapeDtypeStruct((N_PAD, 1), jnp.float32),
        ],
    )(x_pad, W, att_src, att_dst)


def _sc_body(hext_hbm, asrc_hbm, adst_hbm, src_hbm, dst_hbm, out_hbm, den_hbm,
             asrc_l, adst_l, src_a, dst_a, src_b, dst_b,
             rows_a, rows_b, ex_a, ex_b, acc_sh, den_sh, sem_a, sem_b):
    c = lax.axis_index("c")
    s = lax.axis_index("s")
    wid = s * NC + c

    pltpu.sync_copy(asrc_hbm, asrc_l)
    pltpu.sync_copy(adst_hbm, adst_l)

    @pl.loop(0, C)
    def _(e):
        for j in range(DH // L):
            rows_a[e, pl.ds(j * L, L)] = jnp.zeros((L,), jnp.float32)

    for g in range(C // L):
        ex_a[pl.ds(g * L, L)] = jnp.zeros((L,), jnp.float32)

    for k in range(ROWS_PER_SUB // C):
        pltpu.sync_copy(rows_a, acc_sh.at[pl.ds(s * ROWS_PER_SUB + k * C, C)])
        pltpu.sync_copy(ex_a, den_sh.at[pl.ds(s * ROWS_PER_SUB + k * C, C)])
    plsc.subcore_barrier()

    base = wid * EDGES_PER_TILE

    def load_idx(k, sv, dv):
        off = base + k * C
        pltpu.sync_copy(src_hbm.at[pl.ds(off, C)], sv)
        pltpu.sync_copy(dst_hbm.at[pl.ds(off, C)], dv)

    dnums = lax.GatherDimensionNumbers(
        offset_dims=(), collapsed_slice_dims=(0,), start_index_map=(0,))

    def process(sv, dv, rows, exv):
        for g in range(C // L):
            sl = pl.ds(g * L, L)
            si = sv[sl]
            di = dv[sl]
            av = plsc.load_gather(asrc_l, [si]) + plsc.load_gather(adst_l, [di])
            av = jnp.maximum(av, av * 0.2)
            exg = jnp.exp(av)
            exv[sl] = exg
            for t in range(L):
                bc = lax.gather(
                    exg, jnp.full((L, 1), t, jnp.int32), dnums, (1,),
                    mode=lax.GatherScatterMode.PROMISE_IN_BOUNDS)
                row = g * L + t
                for j in range(DH // L):
                    sl2 = pl.ds(j * L, L)
                    rows[row, sl2] = rows[row, sl2] * bc
        pltpu.sync_copy(rows, acc_sh.at[dv], add=True)
        pltpu.sync_copy(exv, den_sh.at[dv], add=True)

    load_idx(0, src_a, dst_a)
    pltpu.async_copy(hext_hbm.at[src_a], rows_a, sem_a)

    @pl.loop(0, CHUNKS // 2)
    def _(k2):
        k = 2 * k2
        load_idx(k + 1, src_b, dst_b)
        pltpu.async_copy(hext_hbm.at[src_b], rows_b, sem_b)

        pltpu.make_async_copy(hext_hbm.at[src_a], rows_a, sem_a).wait()
        process(src_a, dst_a, rows_a, ex_a)

        @pl.when(k2 + 1 < CHUNKS // 2)
        def _():
            load_idx(k + 2, src_a, dst_a)
            pltpu.async_copy(hext_hbm.at[src_a], rows_a, sem_a)

        pltpu.make_async_copy(hext_hbm.at[src_b], rows_b, sem_b).wait()
        process(src_b, dst_b, rows_b, ex_b)

    plsc.subcore_barrier()
    pltpu.sync_copy(acc_sh.at[pl.ds(s * ROWS_PER_SUB, ROWS_PER_SUB)],
                    out_hbm.at[c, pl.ds(s * ROWS_PER_SUB, ROWS_PER_SUB)])
    pltpu.sync_copy(den_sh.at[pl.ds(s * ROWS_PER_SUB, ROWS_PER_SUB)],
                    den_hbm.at[c, pl.ds(s * ROWS_PER_SUB, ROWS_PER_SUB)])


def _sc_scatter(h_ext, asrc, adst, src_all, dst_all):
    mesh = plsc.VectorSubcoreMesh(core_axis_name="c", subcore_axis_name="s")
    cp = pltpu.CompilerParams(needs_layout_passes=False,
                              use_tc_tiling_on_sc=False)
    f = pl.kernel(
        _sc_body,
        compiler_params=cp,
        out_type=[
            jax.ShapeDtypeStruct((NC, N_PAD, DH), jnp.float32),
            jax.ShapeDtypeStruct((NC, N_PAD), jnp.float32),
        ],
        mesh=mesh,
        scratch_types=[
            pltpu.VMEM((N_PAD,), jnp.float32),
            pltpu.VMEM((N_PAD,), jnp.float32),
            pltpu.VMEM((C,), jnp.int32),
            pltpu.VMEM((C,), jnp.int32),
            pltpu.VMEM((C,), jnp.int32),
            pltpu.VMEM((C,), jnp.int32),
            pltpu.VMEM((C, DH), jnp.float32),
            pltpu.VMEM((C, DH), jnp.float32),
            pltpu.VMEM((C,), jnp.float32),
            pltpu.VMEM((C,), jnp.float32),
            pltpu.VMEM_SHARED((N_PAD, DH), jnp.float32),
            pltpu.VMEM_SHARED((N_PAD,), jnp.float32),
            pltpu.SemaphoreType.DMA,
            pltpu.SemaphoreType.DMA,
        ],
    )
    return f(h_ext, asrc, adst, src_all, dst_all)


def _final_body(p0_ref, p1_ref, d0_ref, d1_ref, b_ref, o_ref):
    num = p0_ref[...] + p1_ref[...]
    den = d0_ref[...] + d1_ref[...]
    o = num / (den + 1e-16) + b_ref[...]
    o_ref[...] = jnp.where(o > 0, o, jnp.exp(o) - 1.0)


def _finalize(p0, p1, d0, d1, bias):
    blk = 1000
    grid = N // blk
    return pl.pallas_call(
        _final_body,
        grid=(grid,),
        in_specs=[
            pl.BlockSpec((blk, DH), lambda i: (i, 0)),
            pl.BlockSpec((blk, DH), lambda i: (i, 0)),
            pl.BlockSpec((blk, 1), lambda i: (i, 0)),
            pl.BlockSpec((blk, 1), lambda i: (i, 0)),
            pl.BlockSpec((1, D_OUT), lambda i: (0, 0)),
        ],
        out_specs=pl.BlockSpec((blk, D_OUT), lambda i: (i, 0)),
        out_shape=jax.ShapeDtypeStruct((N, D_OUT), jnp.float32),
    )(p0, p1, d0, d1, bias)


def kernel(x, edge_index, W, att_src, att_dst, bias):
    loop = jnp.arange(N, dtype=jnp.int32)
    pad = jnp.full((E_PAD - E_TOT,), N, dtype=jnp.int32)
    src_all = jnp.concatenate([edge_index[0], loop, pad])
    dst_all = jnp.concatenate([edge_index[1], loop, pad])

    x_pad = jnp.concatenate(
        [x, jnp.zeros((N_PAD - N, D_IN), jnp.float32)], axis=0)
    h_ext, sa, sd = _dense_prologue(
        x_pad, W, att_src.reshape(1, D_OUT), att_dst.reshape(1, D_OUT))

    partials, dens = _sc_scatter(h_ext, sa.reshape(N_PAD), sd.reshape(N_PAD),
                                 src_all, dst_all)

    return _finalize(partials[0, :N, :], partials[1, :N, :],
                     dens[0, :N].reshape(N, 1), dens[1, :N].reshape(N, 1),
                     bias.reshape(1, D_OUT))
